# Initial kernel scaffold; baseline (speedup 1.0000x reference)
import jax
import jax.numpy as jnp
from jax.experimental import pallas as pl


def kernel(x, edge_index, Wd, bd, Wg, bg):
    raise NotImplementedError("write your pallas kernel here")


# speedup vs baseline: 3.4552x; 3.4552x over previous
import jax
import jax.numpy as jnp
from jax.experimental import pallas as pl
from jax.experimental.pallas import tpu as pltpu

N = 10000
E = 320000
D = 128

_TM_DENSE = 1000
_TM_POST = 1000
_TM_REC = 400


def _dense_body(x_ref, wd_ref, bd_ref, wg_ref, o_ref):
    v = jnp.dot(x_ref[...], wd_ref[...], preferred_element_type=jnp.float32)
    v = v + bd_ref[...]
    v = jnp.where(v >= 0, v, 0.01 * v)
    o_ref[...] = jnp.dot(v, wg_ref[...], preferred_element_type=jnp.float32)


def _dense(x, Wd, bd, Wg):
    return pl.pallas_call(
        _dense_body,
        grid=(N // _TM_DENSE,),
        in_specs=[
            pl.BlockSpec((_TM_DENSE, D), lambda i: (i, 0)),
            pl.BlockSpec((D, D), lambda i: (0, 0)),
            pl.BlockSpec((1, D), lambda i: (0, 0)),
            pl.BlockSpec((D, D), lambda i: (0, 0)),
        ],
        out_specs=pl.BlockSpec((_TM_DENSE, D), lambda i: (i, 0)),
        out_shape=jax.ShapeDtypeStruct((N, D), jnp.float32),
        compiler_params=pltpu.CompilerParams(
            dimension_semantics=("parallel",)),
    )(x, Wd, bd.reshape(1, D), Wg)


def _post_body(s_ref, hws_ref, dinv_ref, bg_ref, o_ref):
    v = dinv_ref[...] * (s_ref[...] + hws_ref[...]) + bg_ref[...]
    o_ref[...] = jnp.where(v >= 0, v, 0.01 * v)


def _post(s, hws, dinv, bg):
    return pl.pallas_call(
        _post_body,
        grid=(N // _TM_POST,),
        in_specs=[
            pl.BlockSpec((_TM_POST, D), lambda i: (i, 0)),
            pl.BlockSpec((_TM_POST, D), lambda i: (i, 0)),
            pl.BlockSpec((_TM_POST, 1), lambda i: (i, 0)),
            pl.BlockSpec((1, D), lambda i: (0, 0)),
        ],
        out_specs=pl.BlockSpec((_TM_POST, D), lambda i: (i, 0)),
        out_shape=jax.ShapeDtypeStruct((N, D), jnp.float32),
        compiler_params=pltpu.CompilerParams(
            dimension_semantics=("parallel",)),
    )(s, hws, dinv.reshape(N, 1), bg.reshape(1, D))


def _recons_body(hb_ref, ha_ref, o_ref):
    logits = jax.lax.dot_general(
        hb_ref[...], ha_ref[...],
        (((1,), (1,)), ((), ())),
        preferred_element_type=jnp.float32)
    o_ref[...] = 1.0 / (1.0 + jnp.exp(-logits))


def _recons(h):
    return pl.pallas_call(
        _recons_body,
        grid=(N // _TM_REC,),
        in_specs=[
            pl.BlockSpec((_TM_REC, D), lambda i: (i, 0)),
            pl.BlockSpec((N, D), lambda i: (0, 0)),
        ],
        out_specs=pl.BlockSpec((_TM_REC, N), lambda i: (i, 0)),
        out_shape=jax.ShapeDtypeStruct((N, N), jnp.float32),
        compiler_params=pltpu.CompilerParams(
            dimension_semantics=("arbitrary",)),
    )(h, h)


def kernel(x, edge_index, Wd, bd, Wg, bg):
    src = edge_index[0]
    dst = edge_index[1]
    hw = _dense(x, Wd, bd, Wg)
    deg = jnp.zeros((N,), jnp.float32).at[dst].add(1.0) + 1.0
    dinv = jax.lax.rsqrt(deg)
    hws = dinv[:, None] * hw
    s = jnp.zeros((N, D), jnp.float32).at[dst].add(hws[src])
    h = _post(s, hws, dinv, bg)
    return _recons(h)


# speedup vs baseline: 19.8044x; 5.7318x over previous
import functools

import jax
import jax.numpy as jnp
from jax import lax
from jax.experimental import pallas as pl
from jax.experimental.pallas import tpu as pltpu
from jax.experimental.pallas import tpu_sc as plsc

N = 10000
E = 320000
D = 128

_TM_DENSE = 1000
_TM_POST = 1000
_TM_REC = 400

_NC = 2
_NS = 16
_NW = _NC * _NS
_LANES = 128
_EP = 327680
_IDX_ROWS = _EP // _LANES
_RPT = _IDX_ROWS // _NW
_NACC = 10240
_STRIPE = _NACC // _NS


def _dense_body(x_ref, wd_ref, bd_ref, wg_ref, o_ref):
    v = jnp.dot(x_ref[...], wd_ref[...], preferred_element_type=jnp.float32)
    v = v + bd_ref[...]
    v = jnp.where(v >= 0, v, 0.01 * v)
    o_ref[...] = jnp.dot(v, wg_ref[...], preferred_element_type=jnp.float32)


def _dense(x, Wd, bd, Wg):
    return pl.pallas_call(
        _dense_body,
        grid=(N // _TM_DENSE,),
        in_specs=[
            pl.BlockSpec((_TM_DENSE, D), lambda i: (i, 0)),
            pl.BlockSpec((D, D), lambda i: (0, 0)),
            pl.BlockSpec((1, D), lambda i: (0, 0)),
            pl.BlockSpec((D, D), lambda i: (0, 0)),
        ],
        out_specs=pl.BlockSpec((_TM_DENSE, D), lambda i: (i, 0)),
        out_shape=jax.ShapeDtypeStruct((N, D), jnp.float32),
        compiler_params=pltpu.CompilerParams(
            dimension_semantics=("parallel",)),
    )(x, Wd, bd.reshape(1, D), Wg)


def _sc_deg(dst2d):
    mesh = plsc.VectorSubcoreMesh(core_axis_name="c", subcore_axis_name="s")

    @functools.partial(
        pl.kernel,
        out_type=jax.ShapeDtypeStruct((_NC * _NACC,), jnp.float32),
        mesh=mesh,
        scratch_types=[
            pltpu.VMEM((_RPT, _LANES), jnp.int32),
            pltpu.VMEM((_LANES,), jnp.float32),
            pltpu.VMEM((_LANES,), jnp.float32),
            pltpu.MemorySpace.VMEM_SHARED((_NACC,), jnp.float32),
        ],
    )
    def deg_kernel(dst_hbm, out_hbm, didx, ones_v, zb, dacc):
        c = lax.axis_index("c")
        s = lax.axis_index("s")
        wid = s * _NC + c

        for cb in range(_LANES // 16):
            ones_v[pl.ds(cb * 16, 16)] = jnp.ones((16,), jnp.float32)
        for cb in range(_LANES // 16):
            zb[pl.ds(cb * 16, 16)] = jnp.zeros((16,), jnp.float32)
        for k in range(_STRIPE // _LANES):
            pltpu.sync_copy(zb, dacc.at[pl.ds(s * _STRIPE + k * _LANES,
                                              _LANES)])
        plsc.subcore_barrier()
        pltpu.sync_copy(dst_hbm.at[pl.ds(wid * _RPT, _RPT)], didx)

        def body(j, carry):
            pltpu.sync_copy(ones_v, dacc.at[didx.at[j]], add=True)
            return carry
        lax.fori_loop(0, _RPT, body, 0)
        plsc.subcore_barrier()
        pltpu.sync_copy(dacc.at[pl.ds(s * _STRIPE, _STRIPE)],
                        out_hbm.at[pl.ds(c * _NACC + s * _STRIPE, _STRIPE)])

    return deg_kernel(dst2d).reshape(_NC, _NACC)


def _sc_scatter(hws, src2d, dst2d):
    mesh = plsc.VectorSubcoreMesh(core_axis_name="c", subcore_axis_name="s")

    @functools.partial(
        pl.kernel,
        out_type=jax.ShapeDtypeStruct((_NC, _NACC, D), jnp.float32),
        mesh=mesh,
        scratch_types=[
            pltpu.VMEM((_RPT, _LANES), jnp.int32),
            pltpu.VMEM((_RPT, _LANES), jnp.int32),
            pltpu.VMEM((_LANES, D), jnp.float32),
            pltpu.MemorySpace.VMEM_SHARED((_NACC, D), jnp.float32),
            pltpu.SemaphoreType.DMA,
        ],
    )
    def scat_kernel(hws_hbm, src_hbm, dst_hbm, out_hbm,
                    sidx, didx, rows, acc, sem):
        c = lax.axis_index("c")
        s = lax.axis_index("s")
        wid = s * _NC + c

        def zr(i, carry):
            for cb in range(D // 16):
                rows[i, pl.ds(cb * 16, 16)] = jnp.zeros((16,), jnp.float32)
            return carry
        lax.fori_loop(0, _LANES, zr, 0)
        for k in range(_STRIPE // _LANES):
            pltpu.sync_copy(rows, acc.at[pl.ds(s * _STRIPE + k * _LANES,
                                               _LANES)])
        plsc.subcore_barrier()
        pltpu.sync_copy(src_hbm.at[pl.ds(wid * _RPT, _RPT)], sidx)
        pltpu.sync_copy(dst_hbm.at[pl.ds(wid * _RPT, _RPT)], didx)

        def body(j, carry):
            pltpu.async_copy(hws_hbm.at[sidx.at[j]], rows, sem).wait()
            pltpu.sync_copy(rows, acc.at[didx.at[j]], add=True)
            return carry
        lax.fori_loop(0, _RPT, body, 0)
        plsc.subcore_barrier()
        pltpu.sync_copy(acc.at[pl.ds(s * _STRIPE, _STRIPE)],
                        out_hbm.at[c, pl.ds(s * _STRIPE, _STRIPE)])

    return scat_kernel(hws, src2d, dst2d)


def _scale_body(hw_ref, d0_ref, d1_ref, hws_ref, dinv_ref):
    deg = d0_ref[...] + d1_ref[...] + 1.0
    dinv = lax.rsqrt(deg)
    dinv_ref[...] = dinv
    hws_ref[...] = dinv * hw_ref[...]


def _scale(hw, d0, d1):
    return pl.pallas_call(
        _scale_body,
        grid=(N // _TM_POST,),
        in_specs=[
            pl.BlockSpec((_TM_POST, D), lambda i: (i, 0)),
            pl.BlockSpec((_TM_POST, 1), lambda i: (i, 0)),
            pl.BlockSpec((_TM_POST, 1), lambda i: (i, 0)),
        ],
        out_specs=[
            pl.BlockSpec((_TM_POST, D), lambda i: (i, 0)),
            pl.BlockSpec((_TM_POST, 1), lambda i: (i, 0)),
        ],
        out_shape=[
            jax.ShapeDtypeStruct((N, D), jnp.float32),
            jax.ShapeDtypeStruct((N, 1), jnp.float32),
        ],
        compiler_params=pltpu.CompilerParams(
            dimension_semantics=("parallel",)),
    )(hw, d0, d1)


def _post_body(sp_ref, hws_ref, dinv_ref, bg_ref, o_ref):
    v = dinv_ref[...] * (sp_ref[0] + sp_ref[1] + hws_ref[...]) + bg_ref[...]
    o_ref[...] = jnp.where(v >= 0, v, 0.01 * v)


def _post(spart, hws, dinv, bg):
    return pl.pallas_call(
        _post_body,
        grid=(N // _TM_POST,),
        in_specs=[
            pl.BlockSpec((_NC, _TM_POST, D), lambda i: (0, i, 0)),
            pl.BlockSpec((_TM_POST, D), lambda i: (i, 0)),
            pl.BlockSpec((_TM_POST, 1), lambda i: (i, 0)),
            pl.BlockSpec((1, D), lambda i: (0, 0)),
        ],
        out_specs=pl.BlockSpec((_TM_POST, D), lambda i: (i, 0)),
        out_shape=jax.ShapeDtypeStruct((N, D), jnp.float32),
        compiler_params=pltpu.CompilerParams(
            dimension_semantics=("parallel",)),
    )(spart, hws, dinv, bg.reshape(1, D))


def _recons_body(hb_ref, ha_ref, o_ref):
    logits = lax.dot_general(
        hb_ref[...], ha_ref[...],
        (((1,), (1,)), ((), ())),
        preferred_element_type=jnp.float32)
    o_ref[...] = 1.0 / (1.0 + jnp.exp(-logits))


def _recons(h):
    return pl.pallas_call(
        _recons_body,
        grid=(N // _TM_REC,),
        in_specs=[
            pl.BlockSpec((_TM_REC, D), lambda i: (i, 0)),
            pl.BlockSpec((N, D), lambda i: (0, 0)),
        ],
        out_specs=pl.BlockSpec((_TM_REC, N), lambda i: (i, 0)),
        out_shape=jax.ShapeDtypeStruct((N, N), jnp.float32),
        compiler_params=pltpu.CompilerParams(
            dimension_semantics=("arbitrary",)),
    )(h, h)


def kernel(x, edge_index, Wd, bd, Wg, bg):
    src = edge_index[0]
    dst = edge_index[1]
    pad = _EP - E
    ar = jnp.arange(pad, dtype=jnp.int32)
    src2d = jnp.concatenate([src, ar % 64]).reshape(_IDX_ROWS, _LANES)
    dst2d = jnp.concatenate([dst, N + (ar % 16)]).reshape(_IDX_ROWS, _LANES)

    hw = _dense(x, Wd, bd, Wg)
    degp = _sc_deg(dst2d)
    d0 = degp[0, :N].reshape(N, 1)
    d1 = degp[1, :N].reshape(N, 1)
    hws, dinv = _scale(hw, d0, d1)
    spart = _sc_scatter(hws, src2d, dst2d)
    h = _post(spart, hws, dinv, bg)
    return _recons(h)


# speedup vs baseline: 20.8974x; 1.0552x over previous
import functools

import jax
import jax.numpy as jnp
from jax import lax
from jax.experimental import pallas as pl
from jax.experimental.pallas import tpu as pltpu
from jax.experimental.pallas import tpu_sc as plsc

N = 10000
E = 320000
D = 128

_TM_DENSE = 1000
_TM_POST = 1000
_TM_REC = 400

_NC = 2
_NS = 16
_NW = _NC * _NS
_LANES = 128
_EP = 327680
_IDX_ROWS = _EP // _LANES
_RPT = _IDX_ROWS // _NW
_NACC = 10240
_STRIPE = _NACC // _NS
_HD = D // 2
_RPC = _IDX_ROWS // _NS


def _dense_body(x_ref, wd_ref, bd_ref, wg_ref, o_ref):
    v = jnp.dot(x_ref[...], wd_ref[...], preferred_element_type=jnp.float32)
    v = v + bd_ref[...]
    v = jnp.where(v >= 0, v, 0.01 * v)
    o_ref[...] = jnp.dot(v, wg_ref[...], preferred_element_type=jnp.float32)


def _dense(x, Wd, bd, Wg):
    return pl.pallas_call(
        _dense_body,
        grid=(N // _TM_DENSE,),
        in_specs=[
            pl.BlockSpec((_TM_DENSE, D), lambda i: (i, 0)),
            pl.BlockSpec((D, D), lambda i: (0, 0)),
            pl.BlockSpec((1, D), lambda i: (0, 0)),
            pl.BlockSpec((D, D), lambda i: (0, 0)),
        ],
        out_specs=pl.BlockSpec((_TM_DENSE, D), lambda i: (i, 0)),
        out_shape=jax.ShapeDtypeStruct((N, D), jnp.float32),
        compiler_params=pltpu.CompilerParams(
            dimension_semantics=("parallel",)),
    )(x, Wd, bd.reshape(1, D), Wg)


def _sc_deg(dst2d):
    mesh = plsc.VectorSubcoreMesh(core_axis_name="c", subcore_axis_name="s")

    @functools.partial(
        pl.kernel,
        out_type=jax.ShapeDtypeStruct((_NC * _NACC,), jnp.float32),
        mesh=mesh,
        scratch_types=[
            pltpu.VMEM((_RPT, _LANES), jnp.int32),
            pltpu.VMEM((_LANES,), jnp.float32),
            pltpu.VMEM((_LANES,), jnp.float32),
            pltpu.MemorySpace.VMEM_SHARED((_NACC,), jnp.float32),
        ],
    )
    def deg_kernel(dst_hbm, out_hbm, didx, ones_v, zb, dacc):
        c = lax.axis_index("c")
        s = lax.axis_index("s")
        wid = s * _NC + c

        for cb in range(_LANES // 16):
            ones_v[pl.ds(cb * 16, 16)] = jnp.ones((16,), jnp.float32)
        for cb in range(_LANES // 16):
            zb[pl.ds(cb * 16, 16)] = jnp.zeros((16,), jnp.float32)
        for k in range(_STRIPE // _LANES):
            pltpu.sync_copy(zb, dacc.at[pl.ds(s * _STRIPE + k * _LANES,
                                              _LANES)])
        plsc.subcore_barrier()
        pltpu.sync_copy(dst_hbm.at[pl.ds(wid * _RPT, _RPT)], didx)

        def body(j, carry):
            pltpu.sync_copy(ones_v, dacc.at[didx.at[j]], add=True)
            return carry
        lax.fori_loop(0, _RPT, body, 0)
        plsc.subcore_barrier()
        pltpu.sync_copy(dacc.at[pl.ds(s * _STRIPE, _STRIPE)],
                        out_hbm.at[pl.ds(c * _NACC + s * _STRIPE, _STRIPE)])

    return deg_kernel(dst2d).reshape(_NC, _NACC)


def _sc_scatter(hws2, srcoff, dst2d):
    mesh = plsc.VectorSubcoreMesh(core_axis_name="c", subcore_axis_name="s")

    @functools.partial(
        pl.kernel,
        out_type=jax.ShapeDtypeStruct((_NC, _NACC, _HD), jnp.float32),
        mesh=mesh,
        scratch_types=[
            pltpu.VMEM((_RPC, _LANES), jnp.int32),
            pltpu.VMEM((_RPC, _LANES), jnp.int32),
            pltpu.VMEM((_LANES, _HD), jnp.float32),
            pltpu.VMEM((_LANES, _HD), jnp.float32),
            pltpu.MemorySpace.VMEM_SHARED((_NACC, _HD), jnp.float32),
            pltpu.SemaphoreType.DMA,
        ],
        compiler_params=pltpu.CompilerParams(use_tc_tiling_on_sc=False),
    )
    def scat_kernel(hws_hbm, src_hbm, dst_hbm, out_hbm,
                    sidx, didx, rows_a, rows_b, acc, sem_a):
        c = lax.axis_index("c")
        s = lax.axis_index("s")
        bufs = (rows_a, rows_b)

        def zr(i, carry):
            for cb in range(_HD // 16):
                rows_a[i, pl.ds(cb * 16, 16)] = jnp.zeros((16,), jnp.float32)
            return carry
        lax.fori_loop(0, _LANES, zr, 0)

        def zcopy(k, carry):
            pltpu.sync_copy(rows_a,
                            acc.at[pl.ds(s * _STRIPE + k * _LANES, _LANES)])
            return carry
        lax.fori_loop(0, _STRIPE // _LANES, zcopy, 0)
        plsc.subcore_barrier()
        pltpu.sync_copy(src_hbm.at[c, pl.ds(s * _RPC, _RPC)], sidx)
        pltpu.sync_copy(dst_hbm.at[pl.ds(s * _RPC, _RPC)], didx)

        for b in range(2):
            pltpu.async_copy(hws_hbm.at[sidx.at[b]], bufs[b], sem_a)

        def body(i, carry):
            for b in range(2):
                j = 2 * i + b
                pltpu.make_async_copy(
                    hws_hbm.at[sidx.at[j]], bufs[b], sem_a).wait()
                pltpu.sync_copy(bufs[b], acc.at[didx.at[j]], add=True)
                pltpu.async_copy(hws_hbm.at[sidx.at[j + 2]], bufs[b], sem_a)
            return carry
        lax.fori_loop(0, _RPC // 2 - 1, body, 0)
        for b in range(2):
            jj = _RPC - 2 + b
            pltpu.make_async_copy(
                hws_hbm.at[sidx.at[jj]], bufs[b], sem_a).wait()
            pltpu.sync_copy(bufs[b], acc.at[didx.at[jj]], add=True)
        plsc.subcore_barrier()
        pltpu.sync_copy(acc.at[pl.ds(s * _STRIPE, _STRIPE)],
                        out_hbm.at[c, pl.ds(s * _STRIPE, _STRIPE)])

    return scat_kernel(hws2, srcoff, dst2d)


def _scale_body(hw_ref, d0_ref, d1_ref, hws_ref, dinv_ref):
    deg = d0_ref[...] + d1_ref[...] + 1.0
    dinv = lax.rsqrt(deg)
    dinv_ref[...] = dinv
    hws_ref[...] = dinv * hw_ref[...]


def _scale(hw, d0, d1):
    return pl.pallas_call(
        _scale_body,
        grid=(N // _TM_POST,),
        in_specs=[
            pl.BlockSpec((_TM_POST, D), lambda i: (i, 0)),
            pl.BlockSpec((_TM_POST, 1), lambda i: (i, 0)),
            pl.BlockSpec((_TM_POST, 1), lambda i: (i, 0)),
        ],
        out_specs=[
            pl.BlockSpec((_TM_POST, D), lambda i: (i, 0)),
            pl.BlockSpec((_TM_POST, 1), lambda i: (i, 0)),
        ],
        out_shape=[
            jax.ShapeDtypeStruct((N, D), jnp.float32),
            jax.ShapeDtypeStruct((N, 1), jnp.float32),
        ],
        compiler_params=pltpu.CompilerParams(
            dimension_semantics=("parallel",)),
    )(hw, d0, d1)


def _post_body(sp_ref, hws_ref, dinv_ref, bg_ref, o_ref):
    sfull = jnp.concatenate([sp_ref[0], sp_ref[1]], axis=1)
    v = dinv_ref[...] * (sfull + hws_ref[...]) + bg_ref[...]
    o_ref[...] = jnp.where(v >= 0, v, 0.01 * v)


def _post(spart, hws, dinv, bg):
    return pl.pallas_call(
        _post_body,
        grid=(N // _TM_POST,),
        in_specs=[
            pl.BlockSpec((_NC, _TM_POST, _HD), lambda i: (0, i, 0)),
            pl.BlockSpec((_TM_POST, D), lambda i: (i, 0)),
            pl.BlockSpec((_TM_POST, 1), lambda i: (i, 0)),
            pl.BlockSpec((1, D), lambda i: (0, 0)),
        ],
        out_specs=pl.BlockSpec((_TM_POST, D), lambda i: (i, 0)),
        out_shape=jax.ShapeDtypeStruct((N, D), jnp.float32),
        compiler_params=pltpu.CompilerParams(
            dimension_semantics=("parallel",)),
    )(spart, hws, dinv, bg.reshape(1, D))


def _recons_body(hb_ref, ha_ref, o_ref):
    logits = lax.dot_general(
        hb_ref[...], ha_ref[...],
        (((1,), (1,)), ((), ())),
        preferred_element_type=jnp.float32)
    o_ref[...] = 1.0 / (1.0 + jnp.exp(-logits))


def _recons(h):
    return pl.pallas_call(
        _recons_body,
        grid=(N // _TM_REC,),
        in_specs=[
            pl.BlockSpec((_TM_REC, D), lambda i: (i, 0)),
            pl.BlockSpec((N, D), lambda i: (0, 0)),
        ],
        out_specs=pl.BlockSpec((_TM_REC, N), lambda i: (i, 0)),
        out_shape=jax.ShapeDtypeStruct((N, N), jnp.float32),
        compiler_params=pltpu.CompilerParams(
            dimension_semantics=("arbitrary",)),
    )(h, h)


def kernel(x, edge_index, Wd, bd, Wg, bg):
    src = edge_index[0]
    dst = edge_index[1]
    pad = _EP - E
    ar = jnp.arange(pad, dtype=jnp.int32)
    src2d = jnp.concatenate([src, ar % 64]).reshape(_IDX_ROWS, _LANES)
    dst2d = jnp.concatenate([dst, N + (ar % 16)]).reshape(_IDX_ROWS, _LANES)

    hw = _dense(x, Wd, bd, Wg)
    degp = _sc_deg(dst2d)
    d0 = degp[0, :N].reshape(N, 1)
    d1 = degp[1, :N].reshape(N, 1)
    hws, dinv = _scale(hw, d0, d1)
    hws2 = jnp.concatenate([hws[:, :_HD], hws[:, _HD:]], axis=0)
    srcoff = jnp.stack([src2d, src2d + N])
    spart = _sc_scatter(hws2, srcoff, dst2d)
    h = _post(spart, hws, dinv, bg)
    return _recons(h)


# speedup vs baseline: 24.2878x; 1.1622x over previous
import functools

import jax
import jax.numpy as jnp
from jax import lax
from jax.experimental import pallas as pl
from jax.experimental.pallas import tpu as pltpu
from jax.experimental.pallas import tpu_sc as plsc

N = 10000
E = 320000
D = 128

_TM_DENSE = 1000
_TM_POST = 1000
_TM_REC = 400

_NC = 2
_NS = 16
_NW = _NC * _NS
_LANES = 128
_EP = 327680
_IDX_ROWS = _EP // _LANES
_RPT = _IDX_ROWS // _NW
_NACC = 10240
_STRIPE = _NACC // _NS
_HD = D // 2
_RPC = _IDX_ROWS // _NS


def _dense_body(x_ref, wd_ref, bd_ref, wg_ref, o_ref):
    v = jnp.dot(x_ref[...], wd_ref[...], preferred_element_type=jnp.float32)
    v = v + bd_ref[...]
    v = jnp.where(v >= 0, v, 0.01 * v)
    o_ref[...] = jnp.dot(v, wg_ref[...], preferred_element_type=jnp.float32)


def _dense(x, Wd, bd, Wg):
    return pl.pallas_call(
        _dense_body,
        grid=(N // _TM_DENSE,),
        in_specs=[
            pl.BlockSpec((_TM_DENSE, D), lambda i: (i, 0)),
            pl.BlockSpec((D, D), lambda i: (0, 0)),
            pl.BlockSpec((1, D), lambda i: (0, 0)),
            pl.BlockSpec((D, D), lambda i: (0, 0)),
        ],
        out_specs=pl.BlockSpec((_TM_DENSE, D), lambda i: (i, 0)),
        out_shape=jax.ShapeDtypeStruct((N, D), jnp.float32),
        compiler_params=pltpu.CompilerParams(
            dimension_semantics=("parallel",)),
    )(x, Wd, bd.reshape(1, D), Wg)


def _sc_deg(dst2d):
    mesh = plsc.VectorSubcoreMesh(core_axis_name="c", subcore_axis_name="s")

    @functools.partial(
        pl.kernel,
        out_type=jax.ShapeDtypeStruct((_NC * _NACC,), jnp.float32),
        mesh=mesh,
        scratch_types=[
            pltpu.VMEM((_RPT, _LANES), jnp.int32),
            pltpu.VMEM((_LANES,), jnp.float32),
            pltpu.VMEM((_LANES,), jnp.float32),
            pltpu.MemorySpace.VMEM_SHARED((_NACC,), jnp.float32),
        ],
    )
    def deg_kernel(dst_hbm, out_hbm, didx, ones_v, zb, dacc):
        c = lax.axis_index("c")
        s = lax.axis_index("s")
        wid = s * _NC + c

        for cb in range(_LANES // 16):
            ones_v[pl.ds(cb * 16, 16)] = jnp.ones((16,), jnp.float32)
        for cb in range(_LANES // 16):
            zb[pl.ds(cb * 16, 16)] = jnp.zeros((16,), jnp.float32)
        for k in range(_STRIPE // _LANES):
            pltpu.sync_copy(zb, dacc.at[pl.ds(s * _STRIPE + k * _LANES,
                                              _LANES)])
        plsc.subcore_barrier()
        pltpu.sync_copy(dst_hbm.at[pl.ds(wid * _RPT, _RPT)], didx)

        def body(j, carry):
            pltpu.sync_copy(ones_v, dacc.at[didx.at[j]], add=True)
            return carry
        lax.fori_loop(0, _RPT, body, 0)
        plsc.subcore_barrier()
        pltpu.sync_copy(dacc.at[pl.ds(s * _STRIPE, _STRIPE)],
                        out_hbm.at[pl.ds(c * _NACC + s * _STRIPE, _STRIPE)])

    return deg_kernel(dst2d).reshape(_NC, _NACC)


def _sc_scatter(hws2, srcoff, dst2d):
    mesh = plsc.VectorSubcoreMesh(core_axis_name="c", subcore_axis_name="s")

    @functools.partial(
        pl.kernel,
        out_type=jax.ShapeDtypeStruct((_NC, _NACC, _HD), jnp.float32),
        mesh=mesh,
        scratch_types=[
            pltpu.VMEM((_RPC, _LANES), jnp.int32),
            pltpu.VMEM((_RPC, _LANES), jnp.int32),
            pltpu.VMEM((_LANES, _HD), jnp.float32),
            pltpu.VMEM((_LANES, _HD), jnp.float32),
            pltpu.VMEM((_LANES, _HD), jnp.float32),
            pltpu.VMEM((_LANES, _HD), jnp.float32),
            pltpu.MemorySpace.VMEM_SHARED((_NACC, _HD), jnp.float32),
            pltpu.SemaphoreType.DMA,
        ],
        compiler_params=pltpu.CompilerParams(use_tc_tiling_on_sc=False),
    )
    def scat_kernel(hws_hbm, src_hbm, dst_hbm, out_hbm,
                    sidx, didx, rows_a, rows_b, rows_c, rows_d, acc, sem_a):
        c = lax.axis_index("c")
        s = lax.axis_index("s")
        bufs = (rows_a, rows_b, rows_c, rows_d)
        nbuf = len(bufs)

        def zr(i, carry):
            for cb in range(_HD // 16):
                rows_a[i, pl.ds(cb * 16, 16)] = jnp.zeros((16,), jnp.float32)
            return carry
        lax.fori_loop(0, _LANES, zr, 0)

        def zcopy(k, carry):
            pltpu.sync_copy(rows_a,
                            acc.at[pl.ds(s * _STRIPE + k * _LANES, _LANES)])
            return carry
        lax.fori_loop(0, _STRIPE // _LANES, zcopy, 0)
        plsc.subcore_barrier()
        pltpu.sync_copy(src_hbm.at[c, pl.ds(s * _RPC, _RPC)], sidx)
        pltpu.sync_copy(dst_hbm.at[pl.ds(s * _RPC, _RPC)], didx)

        for b in range(nbuf):
            pltpu.async_copy(hws_hbm.at[sidx.at[b]], bufs[b], sem_a)

        def body(i, carry):
            for b in range(nbuf):
                j = nbuf * i + b
                pltpu.make_async_copy(
                    hws_hbm.at[sidx.at[j]], bufs[b], sem_a).wait()
                pltpu.sync_copy(bufs[b], acc.at[didx.at[j]], add=True)
                pltpu.async_copy(hws_hbm.at[sidx.at[j + nbuf]], bufs[b],
                                 sem_a)
            return carry
        lax.fori_loop(0, _RPC // nbuf - 1, body, 0)
        for b in range(nbuf):
            jj = _RPC - nbuf + b
            pltpu.make_async_copy(
                hws_hbm.at[sidx.at[jj]], bufs[b], sem_a).wait()
            pltpu.sync_copy(bufs[b], acc.at[didx.at[jj]], add=True)
        plsc.subcore_barrier()
        pltpu.sync_copy(acc.at[pl.ds(s * _STRIPE, _STRIPE)],
                        out_hbm.at[c, pl.ds(s * _STRIPE, _STRIPE)])

    return scat_kernel(hws2, srcoff, dst2d)


def _scale_body(hw_ref, d0_ref, d1_ref, hws_ref, dinv_ref):
    deg = d0_ref[...] + d1_ref[...] + 1.0
    dinv = lax.rsqrt(deg)
    dinv_ref[...] = dinv
    hws_ref[...] = dinv * hw_ref[...]


def _scale(hw, d0, d1):
    return pl.pallas_call(
        _scale_body,
        grid=(N // _TM_POST,),
        in_specs=[
            pl.BlockSpec((_TM_POST, D), lambda i: (i, 0)),
            pl.BlockSpec((_TM_POST, 1), lambda i: (i, 0)),
            pl.BlockSpec((_TM_POST, 1), lambda i: (i, 0)),
        ],
        out_specs=[
            pl.BlockSpec((_TM_POST, D), lambda i: (i, 0)),
            pl.BlockSpec((_TM_POST, 1), lambda i: (i, 0)),
        ],
        out_shape=[
            jax.ShapeDtypeStruct((N, D), jnp.float32),
            jax.ShapeDtypeStruct((N, 1), jnp.float32),
        ],
        compiler_params=pltpu.CompilerParams(
            dimension_semantics=("parallel",)),
    )(hw, d0, d1)


def _post_body(sp_ref, hws_ref, dinv_ref, bg_ref, o_ref):
    sfull = jnp.concatenate([sp_ref[0], sp_ref[1]], axis=1)
    v = dinv_ref[...] * (sfull + hws_ref[...]) + bg_ref[...]
    o_ref[...] = jnp.where(v >= 0, v, 0.01 * v)


def _post(spart, hws, dinv, bg):
    return pl.pallas_call(
        _post_body,
        grid=(N // _TM_POST,),
        in_specs=[
            pl.BlockSpec((_NC, _TM_POST, _HD), lambda i: (0, i, 0)),
            pl.BlockSpec((_TM_POST, D), lambda i: (i, 0)),
            pl.BlockSpec((_TM_POST, 1), lambda i: (i, 0)),
            pl.BlockSpec((1, D), lambda i: (0, 0)),
        ],
        out_specs=pl.BlockSpec((_TM_POST, D), lambda i: (i, 0)),
        out_shape=jax.ShapeDtypeStruct((N, D), jnp.float32),
        compiler_params=pltpu.CompilerParams(
            dimension_semantics=("parallel",)),
    )(spart, hws, dinv, bg.reshape(1, D))


def _recons_body(hb_ref, ha_ref, o_ref):
    logits = lax.dot_general(
        hb_ref[...], ha_ref[...],
        (((1,), (1,)), ((), ())),
        preferred_element_type=jnp.float32)
    o_ref[...] = 0.5 * jnp.tanh(0.5 * logits) + 0.5


def _recons(h):
    return pl.pallas_call(
        _recons_body,
        grid=(N // _TM_REC,),
        in_specs=[
            pl.BlockSpec((_TM_REC, D), lambda i: (i, 0)),
            pl.BlockSpec((N, D), lambda i: (0, 0)),
        ],
        out_specs=pl.BlockSpec((_TM_REC, N), lambda i: (i, 0)),
        out_shape=jax.ShapeDtypeStruct((N, N), jnp.float32),
        compiler_params=pltpu.CompilerParams(
            dimension_semantics=("arbitrary",)),
    )(h, h)


def kernel(x, edge_index, Wd, bd, Wg, bg):
    src = edge_index[0]
    dst = edge_index[1]
    pad = _EP - E
    ar = jnp.arange(pad, dtype=jnp.int32)
    src2d = jnp.concatenate([src, ar % 64]).reshape(_IDX_ROWS, _LANES)
    dst2d = jnp.concatenate([dst, N + (ar % 16)]).reshape(_IDX_ROWS, _LANES)

    hw = _dense(x, Wd, bd, Wg)
    degp = _sc_deg(dst2d)
    d0 = degp[0, :N].reshape(N, 1)
    d1 = degp[1, :N].reshape(N, 1)
    hws, dinv = _scale(hw, d0, d1)
    hws2 = jnp.concatenate([hws[:, :_HD], hws[:, _HD:]], axis=0)
    srcoff = jnp.stack([src2d, src2d + N])
    spart = _sc_scatter(hws2, srcoff, dst2d)
    h = _post(spart, hws, dinv, bg)
    return _recons(h)


# speedup vs baseline: 24.3071x; 1.0008x over previous
import functools

import jax
import jax.numpy as jnp
from jax import lax
from jax.experimental import pallas as pl
from jax.experimental.pallas import tpu as pltpu
from jax.experimental.pallas import tpu_sc as plsc

N = 10000
E = 320000
D = 128

_TM_DENSE = 1000
_TM_POST = 1000
_TM_REC = 400

_NC = 2
_NS = 16
_NW = _NC * _NS
_LANES = 128
_EP = 327680
_IDX_ROWS = _EP // _LANES
_RPT = _IDX_ROWS // _NW
_NACC = 10240
_STRIPE = _NACC // _NS
_HD = D // 2
_RPC = _IDX_ROWS // _NS


def _dense_body(x_ref, wd_ref, bd_ref, wg_ref, o_ref):
    v = jnp.dot(x_ref[...], wd_ref[...], preferred_element_type=jnp.float32)
    v = v + bd_ref[...]
    v = jnp.where(v >= 0, v, 0.01 * v)
    o_ref[...] = jnp.dot(v, wg_ref[...], preferred_element_type=jnp.float32)


def _dense(x, Wd, bd, Wg):
    return pl.pallas_call(
        _dense_body,
        grid=(N // _TM_DENSE,),
        in_specs=[
            pl.BlockSpec((_TM_DENSE, D), lambda i: (i, 0)),
            pl.BlockSpec((D, D), lambda i: (0, 0)),
            pl.BlockSpec((1, D), lambda i: (0, 0)),
            pl.BlockSpec((D, D), lambda i: (0, 0)),
        ],
        out_specs=pl.BlockSpec((_TM_DENSE, D), lambda i: (i, 0)),
        out_shape=jax.ShapeDtypeStruct((N, D), jnp.float32),
        compiler_params=pltpu.CompilerParams(
            dimension_semantics=("parallel",)),
    )(x, Wd, bd.reshape(1, D), Wg)


def _sc_deg(dst2d):
    mesh = plsc.VectorSubcoreMesh(core_axis_name="c", subcore_axis_name="s")

    @functools.partial(
        pl.kernel,
        out_type=jax.ShapeDtypeStruct((_NC * _NACC,), jnp.float32),
        mesh=mesh,
        scratch_types=[
            pltpu.VMEM((_RPT, _LANES), jnp.int32),
            pltpu.VMEM((_LANES,), jnp.float32),
            pltpu.VMEM((_LANES,), jnp.float32),
            pltpu.MemorySpace.VMEM_SHARED((_NACC,), jnp.float32),
        ],
    )
    def deg_kernel(dst_hbm, out_hbm, didx, ones_v, zb, dacc):
        c = lax.axis_index("c")
        s = lax.axis_index("s")
        wid = s * _NC + c

        for cb in range(_LANES // 16):
            ones_v[pl.ds(cb * 16, 16)] = jnp.ones((16,), jnp.float32)
        for cb in range(_LANES // 16):
            zb[pl.ds(cb * 16, 16)] = jnp.zeros((16,), jnp.float32)
        for k in range(_STRIPE // _LANES):
            pltpu.sync_copy(zb, dacc.at[pl.ds(s * _STRIPE + k * _LANES,
                                              _LANES)])
        plsc.subcore_barrier()
        pltpu.sync_copy(dst_hbm.at[pl.ds(wid * _RPT, _RPT)], didx)

        def body(j, carry):
            pltpu.sync_copy(ones_v, dacc.at[didx.at[j]], add=True)
            return carry
        lax.fori_loop(0, _RPT, body, 0)
        plsc.subcore_barrier()
        pltpu.sync_copy(dacc.at[pl.ds(s * _STRIPE, _STRIPE)],
                        out_hbm.at[pl.ds(c * _NACC + s * _STRIPE, _STRIPE)])

    return deg_kernel(dst2d).reshape(_NC, _NACC)


def _sc_scatter(hws2, srcoff, dst2d):
    mesh = plsc.VectorSubcoreMesh(core_axis_name="c", subcore_axis_name="s")

    @functools.partial(
        pl.kernel,
        out_type=jax.ShapeDtypeStruct((_NC, _NACC, _HD), jnp.float32),
        mesh=mesh,
        scratch_types=[
            pltpu.VMEM((_RPC, _LANES), jnp.int32),
            pltpu.VMEM((_RPC, _LANES), jnp.int32),
            pltpu.VMEM((_LANES, _HD), jnp.float32),
            pltpu.VMEM((_LANES, _HD), jnp.float32),
            pltpu.VMEM((_LANES, _HD), jnp.float32),
            pltpu.VMEM((_LANES, _HD), jnp.float32),
            pltpu.VMEM((_LANES, _HD), jnp.float32),
            pltpu.MemorySpace.VMEM_SHARED((_NACC, _HD), jnp.float32),
            pltpu.SemaphoreType.DMA,
        ],
        compiler_params=pltpu.CompilerParams(use_tc_tiling_on_sc=False),
    )
    def scat_kernel(hws_hbm, src_hbm, dst_hbm, out_hbm,
                    sidx, didx, rows_a, rows_b, rows_c, rows_d,
                    rows_e, acc, sem_a):
        c = lax.axis_index("c")
        s = lax.axis_index("s")
        bufs = (rows_a, rows_b, rows_c, rows_d, rows_e)
        nbuf = len(bufs)

        def zr(i, carry):
            for cb in range(_HD // 16):
                rows_a[i, pl.ds(cb * 16, 16)] = jnp.zeros((16,), jnp.float32)
            return carry
        lax.fori_loop(0, _LANES, zr, 0)

        def zcopy(k, carry):
            pltpu.sync_copy(rows_a,
                            acc.at[pl.ds(s * _STRIPE + k * _LANES, _LANES)])
            return carry
        lax.fori_loop(0, _STRIPE // _LANES, zcopy, 0)
        plsc.subcore_barrier()
        pltpu.sync_copy(src_hbm.at[c, pl.ds(s * _RPC, _RPC)], sidx)
        pltpu.sync_copy(dst_hbm.at[pl.ds(s * _RPC, _RPC)], didx)

        for b in range(nbuf):
            pltpu.async_copy(hws_hbm.at[sidx.at[b]], bufs[b], sem_a)

        def body(i, carry):
            for b in range(nbuf):
                j = nbuf * i + b
                pltpu.make_async_copy(
                    hws_hbm.at[sidx.at[j]], bufs[b], sem_a).wait()
                pltpu.sync_copy(bufs[b], acc.at[didx.at[j]], add=True)
                pltpu.async_copy(hws_hbm.at[sidx.at[j + nbuf]], bufs[b],
                                 sem_a)
            return carry
        lax.fori_loop(0, _RPC // nbuf - 1, body, 0)
        for b in range(nbuf):
            jj = _RPC - nbuf + b
            pltpu.make_async_copy(
                hws_hbm.at[sidx.at[jj]], bufs[b], sem_a).wait()
            pltpu.sync_copy(bufs[b], acc.at[didx.at[jj]], add=True)
        plsc.subcore_barrier()
        pltpu.sync_copy(acc.at[pl.ds(s * _STRIPE, _STRIPE)],
                        out_hbm.at[c, pl.ds(s * _STRIPE, _STRIPE)])

    return scat_kernel(hws2, srcoff, dst2d)


def _scale_body(hw_ref, d0_ref, d1_ref, hws_ref, dinv_ref):
    deg = d0_ref[...] + d1_ref[...] + 1.0
    dinv = lax.rsqrt(deg)
    dinv_ref[...] = dinv
    hws_ref[...] = dinv * hw_ref[...]


def _scale(hw, d0, d1):
    return pl.pallas_call(
        _scale_body,
        grid=(N // _TM_POST,),
        in_specs=[
            pl.BlockSpec((_TM_POST, D), lambda i: (i, 0)),
            pl.BlockSpec((_TM_POST, 1), lambda i: (i, 0)),
            pl.BlockSpec((_TM_POST, 1), lambda i: (i, 0)),
        ],
        out_specs=[
            pl.BlockSpec((_TM_POST, D), lambda i: (i, 0)),
            pl.BlockSpec((_TM_POST, 1), lambda i: (i, 0)),
        ],
        out_shape=[
            jax.ShapeDtypeStruct((N, D), jnp.float32),
            jax.ShapeDtypeStruct((N, 1), jnp.float32),
        ],
        compiler_params=pltpu.CompilerParams(
            dimension_semantics=("parallel",)),
    )(hw, d0, d1)


def _post_body(sp_ref, hws_ref, dinv_ref, bg_ref, o_ref):
    sfull = jnp.concatenate([sp_ref[0], sp_ref[1]], axis=1)
    v = dinv_ref[...] * (sfull + hws_ref[...]) + bg_ref[...]
    o_ref[...] = jnp.where(v >= 0, v, 0.01 * v)


def _post(spart, hws, dinv, bg):
    return pl.pallas_call(
        _post_body,
        grid=(N // _TM_POST,),
        in_specs=[
            pl.BlockSpec((_NC, _TM_POST, _HD), lambda i: (0, i, 0)),
            pl.BlockSpec((_TM_POST, D), lambda i: (i, 0)),
            pl.BlockSpec((_TM_POST, 1), lambda i: (i, 0)),
            pl.BlockSpec((1, D), lambda i: (0, 0)),
        ],
        out_specs=pl.BlockSpec((_TM_POST, D), lambda i: (i, 0)),
        out_shape=jax.ShapeDtypeStruct((N, D), jnp.float32),
        compiler_params=pltpu.CompilerParams(
            dimension_semantics=("parallel",)),
    )(spart, hws, dinv, bg.reshape(1, D))


def _recons_body(hb_ref, ha_ref, o_ref):
    logits = lax.dot_general(
        hb_ref[...], ha_ref[...],
        (((1,), (1,)), ((), ())),
        preferred_element_type=jnp.float32)
    o_ref[...] = 0.5 * jnp.tanh(0.5 * logits) + 0.5


def _recons(h):
    return pl.pallas_call(
        _recons_body,
        grid=(N // _TM_REC,),
        in_specs=[
            pl.BlockSpec((_TM_REC, D), lambda i: (i, 0)),
            pl.BlockSpec((N, D), lambda i: (0, 0)),
        ],
        out_specs=pl.BlockSpec((_TM_REC, N), lambda i: (i, 0)),
        out_shape=jax.ShapeDtypeStruct((N, N), jnp.float32),
        compiler_params=pltpu.CompilerParams(
            dimension_semantics=("arbitrary",)),
    )(h, h)


def kernel(x, edge_index, Wd, bd, Wg, bg):
    src = edge_index[0]
    dst = edge_index[1]
    pad = _EP - E
    ar = jnp.arange(pad, dtype=jnp.int32)
    src2d = jnp.concatenate([src, ar % 64]).reshape(_IDX_ROWS, _LANES)
    dst2d = jnp.concatenate([dst, N + (ar % 16)]).reshape(_IDX_ROWS, _LANES)

    hw = _dense(x, Wd, bd, Wg)
    degp = _sc_deg(dst2d)
    d0 = degp[0, :N].reshape(N, 1)
    d1 = degp[1, :N].reshape(N, 1)
    hws, dinv = _scale(hw, d0, d1)
    hws2 = jnp.concatenate([hws[:, :_HD], hws[:, _HD:]], axis=0)
    srcoff = jnp.stack([src2d, src2d + N])
    spart = _sc_scatter(hws2, srcoff, dst2d)
    h = _post(spart, hws, dinv, bg)
    return _recons(h)


# speedup vs baseline: 24.3115x; 1.0002x over previous
import functools

import jax
import jax.numpy as jnp
from jax import lax
from jax.experimental import pallas as pl
from jax.experimental.pallas import tpu as pltpu
from jax.experimental.pallas import tpu_sc as plsc

N = 10000
E = 320000
D = 128

_TM_DENSE = 1000
_TM_POST = 1000
_TM_REC = 400

_NC = 2
_NS = 16
_NW = _NC * _NS
_LANES = 128
_EP = 327680
_IDX_ROWS = _EP // _LANES
_RPT = _IDX_ROWS // _NW
_NACC = 10240
_STRIPE = _NACC // _NS
_HD = D // 2
_RPC = _IDX_ROWS // _NS


def _dense_body(x_ref, wd_ref, bd_ref, wg_ref, o_ref):
    v = jnp.dot(x_ref[...], wd_ref[...], preferred_element_type=jnp.float32)
    v = v + bd_ref[...]
    v = jnp.where(v >= 0, v, 0.01 * v)
    o_ref[...] = jnp.dot(v, wg_ref[...], preferred_element_type=jnp.float32)


def _dense(x, Wd, bd, Wg):
    return pl.pallas_call(
        _dense_body,
        grid=(N // _TM_DENSE,),
        in_specs=[
            pl.BlockSpec((_TM_DENSE, D), lambda i: (i, 0)),
            pl.BlockSpec((D, D), lambda i: (0, 0)),
            pl.BlockSpec((1, D), lambda i: (0, 0)),
            pl.BlockSpec((D, D), lambda i: (0, 0)),
        ],
        out_specs=pl.BlockSpec((_TM_DENSE, D), lambda i: (i, 0)),
        out_shape=jax.ShapeDtypeStruct((N, D), jnp.float32),
        compiler_params=pltpu.CompilerParams(
            dimension_semantics=("parallel",)),
    )(x, Wd, bd.reshape(1, D), Wg)


def _sc_deg(dst2d):
    mesh = plsc.VectorSubcoreMesh(core_axis_name="c", subcore_axis_name="s")

    @functools.partial(
        pl.kernel,
        out_type=jax.ShapeDtypeStruct((_NC * _NACC,), jnp.float32),
        mesh=mesh,
        scratch_types=[
            pltpu.VMEM((_RPT, _LANES), jnp.int32),
            pltpu.VMEM((_LANES,), jnp.float32),
            pltpu.VMEM((_LANES,), jnp.float32),
            pltpu.MemorySpace.VMEM_SHARED((_NACC,), jnp.float32),
            pltpu.SemaphoreType.DMA,
        ],
    )
    def deg_kernel(dst_hbm, out_hbm, didx, ones_v, zb, dacc, sem):
        c = lax.axis_index("c")
        s = lax.axis_index("s")
        wid = s * _NC + c

        for cb in range(_LANES // 16):
            ones_v[pl.ds(cb * 16, 16)] = jnp.ones((16,), jnp.float32)
        for cb in range(_LANES // 16):
            zb[pl.ds(cb * 16, 16)] = jnp.zeros((16,), jnp.float32)
        for k in range(_STRIPE // _LANES):
            pltpu.sync_copy(zb, dacc.at[pl.ds(s * _STRIPE + k * _LANES,
                                              _LANES)])
        plsc.subcore_barrier()
        pltpu.sync_copy(dst_hbm.at[pl.ds(wid * _RPT, _RPT)], didx)

        def body(j, carry):
            pltpu.async_copy(ones_v, dacc.at[didx.at[j]], sem, add=True)
            return carry
        lax.fori_loop(0, _RPT, body, 0)

        def drain(j, carry):
            pltpu.make_async_copy(ones_v, dacc.at[didx.at[j]], sem).wait()
            return carry
        lax.fori_loop(0, _RPT, drain, 0)
        plsc.subcore_barrier()
        pltpu.sync_copy(dacc.at[pl.ds(s * _STRIPE, _STRIPE)],
                        out_hbm.at[pl.ds(c * _NACC + s * _STRIPE, _STRIPE)])

    return deg_kernel(dst2d).reshape(_NC, _NACC)


def _sc_scatter(hws2, srcoff, dst2d):
    mesh = plsc.VectorSubcoreMesh(core_axis_name="c", subcore_axis_name="s")

    @functools.partial(
        pl.kernel,
        out_type=jax.ShapeDtypeStruct((_NC, _NACC, _HD), jnp.float32),
        mesh=mesh,
        scratch_types=[
            pltpu.VMEM((_RPC, _LANES), jnp.int32),
            pltpu.VMEM((_RPC, _LANES), jnp.int32),
            pltpu.VMEM((_LANES, _HD), jnp.float32),
            pltpu.VMEM((_LANES, _HD), jnp.float32),
            pltpu.VMEM((_LANES, _HD), jnp.float32),
            pltpu.VMEM((_LANES, _HD), jnp.float32),
            pltpu.VMEM((_LANES, _HD), jnp.float32),
            pltpu.MemorySpace.VMEM_SHARED((_NACC, _HD), jnp.float32),
            pltpu.SemaphoreType.DMA,
        ],
        compiler_params=pltpu.CompilerParams(use_tc_tiling_on_sc=False),
    )
    def scat_kernel(hws_hbm, src_hbm, dst_hbm, out_hbm,
                    sidx, didx, rows_a, rows_b, rows_c, rows_d,
                    rows_e, acc, sem_a):
        c = lax.axis_index("c")
        s = lax.axis_index("s")
        bufs = (rows_a, rows_b, rows_c, rows_d, rows_e)
        nbuf = len(bufs)

        def zr(i, carry):
            for cb in range(_HD // 16):
                rows_a[i, pl.ds(cb * 16, 16)] = jnp.zeros((16,), jnp.float32)
            return carry
        lax.fori_loop(0, _LANES, zr, 0)

        def zcopy(k, carry):
            pltpu.sync_copy(rows_a,
                            acc.at[pl.ds(s * _STRIPE + k * _LANES, _LANES)])
            return carry
        lax.fori_loop(0, _STRIPE // _LANES, zcopy, 0)
        plsc.subcore_barrier()
        pltpu.sync_copy(src_hbm.at[c, pl.ds(s * _RPC, _RPC)], sidx)
        pltpu.sync_copy(dst_hbm.at[pl.ds(s * _RPC, _RPC)], didx)

        for b in range(nbuf):
            pltpu.async_copy(hws_hbm.at[sidx.at[b]], bufs[b], sem_a)

        def body(i, carry):
            for b in range(nbuf):
                j = nbuf * i + b
                pltpu.make_async_copy(
                    hws_hbm.at[sidx.at[j]], bufs[b], sem_a).wait()
                pltpu.sync_copy(bufs[b], acc.at[didx.at[j]], add=True)
                pltpu.async_copy(hws_hbm.at[sidx.at[j + nbuf]], bufs[b],
                                 sem_a)
            return carry
        lax.fori_loop(0, _RPC // nbuf - 1, body, 0)
        for b in range(nbuf):
            jj = _RPC - nbuf + b
            pltpu.make_async_copy(
                hws_hbm.at[sidx.at[jj]], bufs[b], sem_a).wait()
            pltpu.sync_copy(bufs[b], acc.at[didx.at[jj]], add=True)
        plsc.subcore_barrier()
        pltpu.sync_copy(acc.at[pl.ds(s * _STRIPE, _STRIPE)],
                        out_hbm.at[c, pl.ds(s * _STRIPE, _STRIPE)])

    return scat_kernel(hws2, srcoff, dst2d)


def _scale_body(hw_ref, d0_ref, d1_ref, hws_ref, dinv_ref):
    deg = d0_ref[...] + d1_ref[...] + 1.0
    dinv = lax.rsqrt(deg)
    dinv_ref[...] = dinv
    hws_ref[...] = dinv * hw_ref[...]


def _scale(hw, d0, d1):
    return pl.pallas_call(
        _scale_body,
        grid=(N // _TM_POST,),
        in_specs=[
            pl.BlockSpec((_TM_POST, D), lambda i: (i, 0)),
            pl.BlockSpec((_TM_POST, 1), lambda i: (i, 0)),
            pl.BlockSpec((_TM_POST, 1), lambda i: (i, 0)),
        ],
        out_specs=[
            pl.BlockSpec((_TM_POST, D), lambda i: (i, 0)),
            pl.BlockSpec((_TM_POST, 1), lambda i: (i, 0)),
        ],
        out_shape=[
            jax.ShapeDtypeStruct((N, D), jnp.float32),
            jax.ShapeDtypeStruct((N, 1), jnp.float32),
        ],
        compiler_params=pltpu.CompilerParams(
            dimension_semantics=("parallel",)),
    )(hw, d0, d1)


def _post_body(sp_ref, hws_ref, dinv_ref, bg_ref, o_ref):
    sfull = jnp.concatenate([sp_ref[0], sp_ref[1]], axis=1)
    v = dinv_ref[...] * (sfull + hws_ref[...]) + bg_ref[...]
    o_ref[...] = jnp.where(v >= 0, v, 0.01 * v)


def _post(spart, hws, dinv, bg):
    return pl.pallas_call(
        _post_body,
        grid=(N // _TM_POST,),
        in_specs=[
            pl.BlockSpec((_NC, _TM_POST, _HD), lambda i: (0, i, 0)),
            pl.BlockSpec((_TM_POST, D), lambda i: (i, 0)),
            pl.BlockSpec((_TM_POST, 1), lambda i: (i, 0)),
            pl.BlockSpec((1, D), lambda i: (0, 0)),
        ],
        out_specs=pl.BlockSpec((_TM_POST, D), lambda i: (i, 0)),
        out_shape=jax.ShapeDtypeStruct((N, D), jnp.float32),
        compiler_params=pltpu.CompilerParams(
            dimension_semantics=("parallel",)),
    )(spart, hws, dinv, bg.reshape(1, D))


def _recons_body(hb_ref, ha_ref, o_ref):
    logits = lax.dot_general(
        hb_ref[...], ha_ref[...],
        (((1,), (1,)), ((), ())),
        preferred_element_type=jnp.float32)
    o_ref[...] = 0.5 * jnp.tanh(0.5 * logits) + 0.5


def _recons(h):
    return pl.pallas_call(
        _recons_body,
        grid=(N // _TM_REC,),
        in_specs=[
            pl.BlockSpec((_TM_REC, D), lambda i: (i, 0)),
            pl.BlockSpec((N, D), lambda i: (0, 0)),
        ],
        out_specs=pl.BlockSpec((_TM_REC, N), lambda i: (i, 0)),
        out_shape=jax.ShapeDtypeStruct((N, N), jnp.float32),
        compiler_params=pltpu.CompilerParams(
            dimension_semantics=("arbitrary",)),
    )(h, h)


def kernel(x, edge_index, Wd, bd, Wg, bg):
    src = edge_index[0]
    dst = edge_index[1]
    pad = _EP - E
    ar = jnp.arange(pad, dtype=jnp.int32)
    src2d = jnp.concatenate([src, ar % 64]).reshape(_IDX_ROWS, _LANES)
    dst2d = jnp.concatenate([dst, N + (ar % 16)]).reshape(_IDX_ROWS, _LANES)

    hw = _dense(x, Wd, bd, Wg)
    degp = _sc_deg(dst2d)
    d0 = degp[0, :N].reshape(N, 1)
    d1 = degp[1, :N].reshape(N, 1)
    hws, dinv = _scale(hw, d0, d1)
    hws2 = jnp.concatenate([hws[:, :_HD], hws[:, _HD:]], axis=0)
    srcoff = jnp.stack([src2d, src2d + N])
    spart = _sc_scatter(hws2, srcoff, dst2d)
    h = _post(spart, hws, dinv, bg)
    return _recons(h)


# speedup vs baseline: 24.8315x; 1.0214x over previous
import functools

import jax
import jax.numpy as jnp
from jax import lax
from jax.experimental import pallas as pl
from jax.experimental.pallas import tpu as pltpu
from jax.experimental.pallas import tpu_sc as plsc

N = 10000
E = 320000
D = 128

_TM_DENSE = 1000
_TM_POST = 1000
_TM_REC = 200

_NC = 2
_NS = 16
_NW = _NC * _NS
_LANES = 128
_EP = 327680
_IDX_ROWS = _EP // _LANES
_RPT = _IDX_ROWS // _NW
_NACC = 10240
_STRIPE = _NACC // _NS
_HD = D // 2
_RPC = _IDX_ROWS // _NS


def _dense_body(x_ref, wd_ref, bd_ref, wg_ref, o_ref):
    v = jnp.dot(x_ref[...], wd_ref[...], preferred_element_type=jnp.float32)
    v = v + bd_ref[...]
    v = jnp.where(v >= 0, v, 0.01 * v)
    o_ref[...] = jnp.dot(v, wg_ref[...], preferred_element_type=jnp.float32)


def _dense(x, Wd, bd, Wg):
    return pl.pallas_call(
        _dense_body,
        grid=(N // _TM_DENSE,),
        in_specs=[
            pl.BlockSpec((_TM_DENSE, D), lambda i: (i, 0)),
            pl.BlockSpec((D, D), lambda i: (0, 0)),
            pl.BlockSpec((1, D), lambda i: (0, 0)),
            pl.BlockSpec((D, D), lambda i: (0, 0)),
        ],
        out_specs=pl.BlockSpec((_TM_DENSE, D), lambda i: (i, 0)),
        out_shape=jax.ShapeDtypeStruct((N, D), jnp.float32),
        compiler_params=pltpu.CompilerParams(
            dimension_semantics=("parallel",)),
    )(x, Wd, bd.reshape(1, D), Wg)


def _sc_deg(dst2d):
    mesh = plsc.VectorSubcoreMesh(core_axis_name="c", subcore_axis_name="s")

    @functools.partial(
        pl.kernel,
        out_type=jax.ShapeDtypeStruct((_NC * _NACC,), jnp.float32),
        mesh=mesh,
        scratch_types=[
            pltpu.VMEM((_RPT, _LANES), jnp.int32),
            pltpu.VMEM((_LANES,), jnp.float32),
            pltpu.VMEM((_LANES,), jnp.float32),
            pltpu.MemorySpace.VMEM_SHARED((_NACC,), jnp.float32),
            pltpu.SemaphoreType.DMA,
        ],
    )
    def deg_kernel(dst_hbm, out_hbm, didx, ones_v, zb, dacc, sem):
        c = lax.axis_index("c")
        s = lax.axis_index("s")
        wid = s * _NC + c

        for cb in range(_LANES // 16):
            ones_v[pl.ds(cb * 16, 16)] = jnp.ones((16,), jnp.float32)
        for cb in range(_LANES // 16):
            zb[pl.ds(cb * 16, 16)] = jnp.zeros((16,), jnp.float32)
        for k in range(_STRIPE // _LANES):
            pltpu.sync_copy(zb, dacc.at[pl.ds(s * _STRIPE + k * _LANES,
                                              _LANES)])
        plsc.subcore_barrier()
        pltpu.sync_copy(dst_hbm.at[pl.ds(wid * _RPT, _RPT)], didx)

        def body(j, carry):
            pltpu.async_copy(ones_v, dacc.at[didx.at[j]], sem, add=True)
            return carry
        lax.fori_loop(0, _RPT, body, 0)

        def drain(j, carry):
            pltpu.make_async_copy(ones_v, dacc.at[didx.at[j]], sem).wait()
            return carry
        lax.fori_loop(0, _RPT, drain, 0)
        plsc.subcore_barrier()
        pltpu.sync_copy(dacc.at[pl.ds(s * _STRIPE, _STRIPE)],
                        out_hbm.at[pl.ds(c * _NACC + s * _STRIPE, _STRIPE)])

    return deg_kernel(dst2d).reshape(_NC, _NACC)


def _sc_scatter(hws2, srcoff, dst2d):
    mesh = plsc.VectorSubcoreMesh(core_axis_name="c", subcore_axis_name="s")

    @functools.partial(
        pl.kernel,
        out_type=jax.ShapeDtypeStruct((_NC, _NACC, _HD), jnp.float32),
        mesh=mesh,
        scratch_types=[
            pltpu.VMEM((_RPC, _LANES), jnp.int32),
            pltpu.VMEM((_RPC, _LANES), jnp.int32),
            pltpu.VMEM((_LANES, _HD), jnp.float32),
            pltpu.VMEM((_LANES, _HD), jnp.float32),
            pltpu.VMEM((_LANES, _HD), jnp.float32),
            pltpu.VMEM((_LANES, _HD), jnp.float32),
            pltpu.VMEM((_LANES, _HD), jnp.float32),
            pltpu.MemorySpace.VMEM_SHARED((_NACC, _HD), jnp.float32),
            pltpu.SemaphoreType.DMA,
        ],
        compiler_params=pltpu.CompilerParams(use_tc_tiling_on_sc=False),
    )
    def scat_kernel(hws_hbm, src_hbm, dst_hbm, out_hbm,
                    sidx, didx, rows_a, rows_b, rows_c, rows_d,
                    rows_e, acc, sem_a):
        c = lax.axis_index("c")
        s = lax.axis_index("s")
        bufs = (rows_a, rows_b, rows_c, rows_d, rows_e)
        nbuf = len(bufs)

        def zr(i, carry):
            for cb in range(_HD // 16):
                rows_a[i, pl.ds(cb * 16, 16)] = jnp.zeros((16,), jnp.float32)
            return carry
        lax.fori_loop(0, _LANES, zr, 0)

        def zcopy(k, carry):
            pltpu.sync_copy(rows_a,
                            acc.at[pl.ds(s * _STRIPE + k * _LANES, _LANES)])
            return carry
        lax.fori_loop(0, _STRIPE // _LANES, zcopy, 0)
        plsc.subcore_barrier()
        pltpu.sync_copy(src_hbm.at[c, pl.ds(s * _RPC, _RPC)], sidx)
        pltpu.sync_copy(dst_hbm.at[pl.ds(s * _RPC, _RPC)], didx)

        for b in range(nbuf):
            pltpu.async_copy(hws_hbm.at[sidx.at[b]], bufs[b], sem_a)

        def body(i, carry):
            for b in range(nbuf):
                j = nbuf * i + b
                pltpu.make_async_copy(
                    hws_hbm.at[sidx.at[j]], bufs[b], sem_a).wait()
                pltpu.sync_copy(bufs[b], acc.at[didx.at[j]], add=True)
                pltpu.async_copy(hws_hbm.at[sidx.at[j + nbuf]], bufs[b],
                                 sem_a)
            return carry
        lax.fori_loop(0, _RPC // nbuf - 1, body, 0)
        for b in range(nbuf):
            jj = _RPC - nbuf + b
            pltpu.make_async_copy(
                hws_hbm.at[sidx.at[jj]], bufs[b], sem_a).wait()
            pltpu.sync_copy(bufs[b], acc.at[didx.at[jj]], add=True)
        plsc.subcore_barrier()
        pltpu.sync_copy(acc.at[pl.ds(s * _STRIPE, _STRIPE)],
                        out_hbm.at[c, pl.ds(s * _STRIPE, _STRIPE)])

    return scat_kernel(hws2, srcoff, dst2d)


def _scale_body(hw_ref, d0_ref, d1_ref, hws_ref, dinv_ref):
    deg = d0_ref[...] + d1_ref[...] + 1.0
    dinv = lax.rsqrt(deg)
    dinv_ref[...] = dinv
    hws_ref[...] = dinv * hw_ref[...]


def _scale(hw, d0, d1):
    return pl.pallas_call(
        _scale_body,
        grid=(N // _TM_POST,),
        in_specs=[
            pl.BlockSpec((_TM_POST, D), lambda i: (i, 0)),
            pl.BlockSpec((_TM_POST, 1), lambda i: (i, 0)),
            pl.BlockSpec((_TM_POST, 1), lambda i: (i, 0)),
        ],
        out_specs=[
            pl.BlockSpec((_TM_POST, D), lambda i: (i, 0)),
            pl.BlockSpec((_TM_POST, 1), lambda i: (i, 0)),
        ],
        out_shape=[
            jax.ShapeDtypeStruct((N, D), jnp.float32),
            jax.ShapeDtypeStruct((N, 1), jnp.float32),
        ],
        compiler_params=pltpu.CompilerParams(
            dimension_semantics=("parallel",)),
    )(hw, d0, d1)


def _recons_body(sp_ref, hws_ref, dinv_ref, bg_ref, o_ref, h_scr):
    @pl.when(pl.program_id(0) == 0)
    def _build_h():
        sfull = jnp.concatenate([sp_ref[0, :N, :], sp_ref[1, :N, :]], axis=1)
        v = dinv_ref[...] * (sfull + hws_ref[...]) + bg_ref[...]
        h_scr[...] = jnp.where(v >= 0, v, 0.01 * v)

    i = pl.program_id(0)
    hb = h_scr[pl.ds(i * _TM_REC, _TM_REC), :]
    logits = lax.dot_general(
        hb, h_scr[...],
        (((1,), (1,)), ((), ())),
        preferred_element_type=jnp.float32)
    o_ref[...] = 0.5 * jnp.tanh(0.5 * logits) + 0.5


def _recons(spart, hws, dinv, bg):
    return pl.pallas_call(
        _recons_body,
        grid=(N // _TM_REC,),
        in_specs=[
            pl.BlockSpec((_NC, _NACC, _HD), lambda i: (0, 0, 0)),
            pl.BlockSpec((N, D), lambda i: (0, 0)),
            pl.BlockSpec((N, 1), lambda i: (0, 0)),
            pl.BlockSpec((1, D), lambda i: (0, 0)),
        ],
        out_specs=pl.BlockSpec((_TM_REC, N), lambda i: (i, 0)),
        out_shape=jax.ShapeDtypeStruct((N, N), jnp.float32),
        scratch_shapes=[pltpu.VMEM((N, D), jnp.float32)],
        compiler_params=pltpu.CompilerParams(
            dimension_semantics=("arbitrary",)),
    )(spart, hws, dinv, bg.reshape(1, D))


def kernel(x, edge_index, Wd, bd, Wg, bg):
    src = edge_index[0]
    dst = edge_index[1]
    pad = _EP - E
    ar = jnp.arange(pad, dtype=jnp.int32)
    src2d = jnp.concatenate([src, ar % 64]).reshape(_IDX_ROWS, _LANES)
    dst2d = jnp.concatenate([dst, N + (ar % 16)]).reshape(_IDX_ROWS, _LANES)

    hw = _dense(x, Wd, bd, Wg)
    degp = _sc_deg(dst2d)
    d0 = degp[0, :N].reshape(N, 1)
    d1 = degp[1, :N].reshape(N, 1)
    hws, dinv = _scale(hw, d0, d1)
    hws2 = jnp.concatenate([hws[:, :_HD], hws[:, _HD:]], axis=0)
    srcoff = jnp.stack([src2d, src2d + N])
    spart = _sc_scatter(hws2, srcoff, dst2d)
    return _recons(spart, hws, dinv, bg)


# speedup vs baseline: 25.8906x; 1.0426x over previous
import functools

import jax
import jax.numpy as jnp
from jax import lax
from jax.experimental import pallas as pl
from jax.experimental.pallas import tpu as pltpu
from jax.experimental.pallas import tpu_sc as plsc

N = 10000
E = 320000
D = 128

_TM_DENSE = 1000
_TM_POST = 1000
_TM_REC = 200

_NC = 2
_NS = 16
_NW = _NC * _NS
_LANES = 128
_EP = 327680
_IDX_ROWS = _EP // _LANES
_RPT = _IDX_ROWS // _NW
_NACC = 10240
_STRIPE = _NACC // _NS
_HD = D // 2
_RPC = _IDX_ROWS // _NS


def _dense_body(x_ref, wd_ref, bd_ref, wg_ref, d0_ref, d1_ref,
                hws_ref, dinv_ref):
    v = jnp.dot(x_ref[...], wd_ref[...], preferred_element_type=jnp.float32)
    v = v + bd_ref[...]
    v = jnp.where(v >= 0, v, 0.01 * v)
    hw = jnp.dot(v, wg_ref[...], preferred_element_type=jnp.float32)
    dinv = lax.rsqrt(d0_ref[...] + d1_ref[...] + 1.0)
    dinv_ref[...] = dinv
    hws = dinv * hw
    hws_ref[0] = hws[:, :_HD]
    hws_ref[1] = hws[:, _HD:]


def _dense(x, Wd, bd, Wg, d0, d1):
    return pl.pallas_call(
        _dense_body,
        grid=(N // _TM_DENSE,),
        in_specs=[
            pl.BlockSpec((_TM_DENSE, D), lambda i: (i, 0)),
            pl.BlockSpec((D, D), lambda i: (0, 0)),
            pl.BlockSpec((1, D), lambda i: (0, 0)),
            pl.BlockSpec((D, D), lambda i: (0, 0)),
            pl.BlockSpec((_TM_DENSE, 1), lambda i: (i, 0)),
            pl.BlockSpec((_TM_DENSE, 1), lambda i: (i, 0)),
        ],
        out_specs=[
            pl.BlockSpec((_NC, _TM_DENSE, _HD), lambda i: (0, i, 0)),
            pl.BlockSpec((_TM_DENSE, 1), lambda i: (i, 0)),
        ],
        out_shape=[
            jax.ShapeDtypeStruct((_NC, N, _HD), jnp.float32),
            jax.ShapeDtypeStruct((N, 1), jnp.float32),
        ],
        compiler_params=pltpu.CompilerParams(
            dimension_semantics=("parallel",)),
    )(x, Wd, bd.reshape(1, D), Wg, d0, d1)


def _sc_deg(dst2d):
    mesh = plsc.VectorSubcoreMesh(core_axis_name="c", subcore_axis_name="s")

    @functools.partial(
        pl.kernel,
        out_type=jax.ShapeDtypeStruct((_NC * _NACC,), jnp.float32),
        mesh=mesh,
        scratch_types=[
            pltpu.VMEM((_RPT, _LANES), jnp.int32),
            pltpu.VMEM((_LANES,), jnp.float32),
            pltpu.VMEM((_LANES,), jnp.float32),
            pltpu.MemorySpace.VMEM_SHARED((_NACC,), jnp.float32),
            pltpu.SemaphoreType.DMA,
        ],
    )
    def deg_kernel(dst_hbm, out_hbm, didx, ones_v, zb, dacc, sem):
        c = lax.axis_index("c")
        s = lax.axis_index("s")
        wid = s * _NC + c

        for cb in range(_LANES // 16):
            ones_v[pl.ds(cb * 16, 16)] = jnp.ones((16,), jnp.float32)
        for cb in range(_LANES // 16):
            zb[pl.ds(cb * 16, 16)] = jnp.zeros((16,), jnp.float32)
        for k in range(_STRIPE // _LANES):
            pltpu.sync_copy(zb, dacc.at[pl.ds(s * _STRIPE + k * _LANES,
                                              _LANES)])
        plsc.subcore_barrier()
        pltpu.sync_copy(dst_hbm.at[pl.ds(wid * _RPT, _RPT)], didx)

        def body(j, carry):
            pltpu.async_copy(ones_v, dacc.at[didx.at[j]], sem, add=True)
            return carry
        lax.fori_loop(0, _RPT, body, 0)

        def drain(j, carry):
            pltpu.make_async_copy(ones_v, dacc.at[didx.at[j]], sem).wait()
            return carry
        lax.fori_loop(0, _RPT, drain, 0)
        plsc.subcore_barrier()
        pltpu.sync_copy(dacc.at[pl.ds(s * _STRIPE, _STRIPE)],
                        out_hbm.at[pl.ds(c * _NACC + s * _STRIPE, _STRIPE)])

    return deg_kernel(dst2d).reshape(_NC, _NACC)


def _sc_scatter(hws2, srcoff, dst2d):
    mesh = plsc.VectorSubcoreMesh(core_axis_name="c", subcore_axis_name="s")

    @functools.partial(
        pl.kernel,
        out_type=jax.ShapeDtypeStruct((_NC, _NACC, _HD), jnp.float32),
        mesh=mesh,
        scratch_types=[
            pltpu.VMEM((_RPC, _LANES), jnp.int32),
            pltpu.VMEM((_RPC, _LANES), jnp.int32),
            pltpu.VMEM((_LANES, _HD), jnp.float32),
            pltpu.VMEM((_LANES, _HD), jnp.float32),
            pltpu.VMEM((_LANES, _HD), jnp.float32),
            pltpu.VMEM((_LANES, _HD), jnp.float32),
            pltpu.VMEM((_LANES, _HD), jnp.float32),
            pltpu.MemorySpace.VMEM_SHARED((_NACC, _HD), jnp.float32),
            pltpu.SemaphoreType.DMA,
        ],
        compiler_params=pltpu.CompilerParams(use_tc_tiling_on_sc=False),
    )
    def scat_kernel(hws_hbm, src_hbm, dst_hbm, out_hbm,
                    sidx, didx, rows_a, rows_b, rows_c, rows_d,
                    rows_e, acc, sem_a):
        c = lax.axis_index("c")
        s = lax.axis_index("s")
        bufs = (rows_a, rows_b, rows_c, rows_d, rows_e)
        nbuf = len(bufs)

        def zr(i, carry):
            for cb in range(_HD // 16):
                rows_a[i, pl.ds(cb * 16, 16)] = jnp.zeros((16,), jnp.float32)
            return carry
        lax.fori_loop(0, _LANES, zr, 0)

        def zcopy(k, carry):
            pltpu.sync_copy(rows_a,
                            acc.at[pl.ds(s * _STRIPE + k * _LANES, _LANES)])
            return carry
        lax.fori_loop(0, _STRIPE // _LANES, zcopy, 0)
        plsc.subcore_barrier()
        pltpu.sync_copy(src_hbm.at[c, pl.ds(s * _RPC, _RPC)], sidx)
        pltpu.sync_copy(dst_hbm.at[pl.ds(s * _RPC, _RPC)], didx)

        for b in range(nbuf):
            pltpu.async_copy(hws_hbm.at[sidx.at[b]], bufs[b], sem_a)

        def body(i, carry):
            for b in range(nbuf):
                j = nbuf * i + b
                pltpu.make_async_copy(
                    hws_hbm.at[sidx.at[j]], bufs[b], sem_a).wait()
                pltpu.sync_copy(bufs[b], acc.at[didx.at[j]], add=True)
                pltpu.async_copy(hws_hbm.at[sidx.at[j + nbuf]], bufs[b],
                                 sem_a)
            return carry
        lax.fori_loop(0, _RPC // nbuf - 1, body, 0)
        for b in range(nbuf):
            jj = _RPC - nbuf + b
            pltpu.make_async_copy(
                hws_hbm.at[sidx.at[jj]], bufs[b], sem_a).wait()
            pltpu.sync_copy(bufs[b], acc.at[didx.at[jj]], add=True)
        plsc.subcore_barrier()
        pltpu.sync_copy(acc.at[pl.ds(s * _STRIPE, _STRIPE)],
                        out_hbm.at[c, pl.ds(s * _STRIPE, _STRIPE)])

    return scat_kernel(hws2, srcoff, dst2d)


def _recons_body(sp_ref, hws_ref, dinv_ref, bg_ref, o_ref, h_scr):
    @pl.when(pl.program_id(0) == 0)
    def _build_h():
        sfull = jnp.concatenate([sp_ref[0, :N, :], sp_ref[1, :N, :]], axis=1)
        hws = jnp.concatenate([hws_ref[0], hws_ref[1]], axis=1)
        v = dinv_ref[...] * (sfull + hws) + bg_ref[...]
        h_scr[...] = jnp.where(v >= 0, v, 0.01 * v)

    i = pl.program_id(0)
    hb = h_scr[pl.ds(i * _TM_REC, _TM_REC), :]
    logits = lax.dot_general(
        hb, h_scr[...],
        (((1,), (1,)), ((), ())),
        preferred_element_type=jnp.float32)
    o_ref[...] = 0.5 * jnp.tanh(0.5 * logits) + 0.5


def _recons(spart, hws2s, dinv, bg):
    return pl.pallas_call(
        _recons_body,
        grid=(N // _TM_REC,),
        in_specs=[
            pl.BlockSpec((_NC, _NACC, _HD), lambda i: (0, 0, 0)),
            pl.BlockSpec((_NC, N, _HD), lambda i: (0, 0, 0)),
            pl.BlockSpec((N, 1), lambda i: (0, 0)),
            pl.BlockSpec((1, D), lambda i: (0, 0)),
        ],
        out_specs=pl.BlockSpec((_TM_REC, N), lambda i: (i, 0)),
        out_shape=jax.ShapeDtypeStruct((N, N), jnp.float32),
        scratch_shapes=[pltpu.VMEM((N, D), jnp.float32)],
        compiler_params=pltpu.CompilerParams(
            dimension_semantics=("arbitrary",)),
    )(spart, hws2s, dinv, bg.reshape(1, D))


def kernel(x, edge_index, Wd, bd, Wg, bg):
    src = edge_index[0]
    dst = edge_index[1]
    pad = _EP - E
    ar = jnp.arange(pad, dtype=jnp.int32)
    src2d = jnp.concatenate([src, ar % 64]).reshape(_IDX_ROWS, _LANES)
    dst2d = jnp.concatenate([dst, N + (ar % 16)]).reshape(_IDX_ROWS, _LANES)

    degp = _sc_deg(dst2d)
    d0 = degp[0, :N].reshape(N, 1)
    d1 = degp[1, :N].reshape(N, 1)
    hws2s, dinv = _dense(x, Wd, bd, Wg, d0, d1)
    hws2 = hws2s.reshape(_NC * N, _HD)
    srcoff = jnp.stack([src2d, src2d + N])
    spart = _sc_scatter(hws2, srcoff, dst2d)
    return _recons(spart, hws2s, dinv, bg)


# speedup vs baseline: 25.9002x; 1.0004x over previous
import functools

import jax
import jax.numpy as jnp
from jax import lax
from jax.experimental import pallas as pl
from jax.experimental.pallas import tpu as pltpu
from jax.experimental.pallas import tpu_sc as plsc

N = 10000
E = 320000
D = 128

_TM_DENSE = 1000
_TM_POST = 1000
_TM_REC = 200

_NC = 2
_NS = 16
_NW = _NC * _NS
_LANES = 128
_EP = 327680
_IDX_ROWS = _EP // _LANES
_RPT = _IDX_ROWS // _NW
_NACC = 10240
_STRIPE = _NACC // _NS
_HD = D // 2
_RPC = _IDX_ROWS // _NS


def _dense_body(x_ref, wd_ref, bd_ref, wg_ref, d0_ref, d1_ref,
                hws_ref, dinv_ref):
    v = jnp.dot(x_ref[...], wd_ref[...], preferred_element_type=jnp.float32)
    v = v + bd_ref[...]
    v = jnp.where(v >= 0, v, 0.01 * v)
    hw = jnp.dot(v, wg_ref[...], preferred_element_type=jnp.float32)
    dinv = lax.rsqrt(d0_ref[...] + d1_ref[...] + 1.0)
    dinv_ref[...] = dinv
    hws = dinv * hw
    hws_ref[0] = hws[:, :_HD]
    hws_ref[1] = hws[:, _HD:]


def _dense(x, Wd, bd, Wg, d0, d1):
    return pl.pallas_call(
        _dense_body,
        grid=(N // _TM_DENSE,),
        in_specs=[
            pl.BlockSpec((_TM_DENSE, D), lambda i: (i, 0)),
            pl.BlockSpec((D, D), lambda i: (0, 0)),
            pl.BlockSpec((1, D), lambda i: (0, 0)),
            pl.BlockSpec((D, D), lambda i: (0, 0)),
            pl.BlockSpec((_TM_DENSE, 1), lambda i: (i, 0)),
            pl.BlockSpec((_TM_DENSE, 1), lambda i: (i, 0)),
        ],
        out_specs=[
            pl.BlockSpec((_NC, _TM_DENSE, _HD), lambda i: (0, i, 0)),
            pl.BlockSpec((_TM_DENSE, 1), lambda i: (i, 0)),
        ],
        out_shape=[
            jax.ShapeDtypeStruct((_NC, N, _HD), jnp.float32),
            jax.ShapeDtypeStruct((N, 1), jnp.float32),
        ],
        compiler_params=pltpu.CompilerParams(
            dimension_semantics=("parallel",)),
    )(x, Wd, bd.reshape(1, D), Wg, d0, d1)


def _sc_deg(dst2d):
    mesh = plsc.VectorSubcoreMesh(core_axis_name="c", subcore_axis_name="s")

    @functools.partial(
        pl.kernel,
        out_type=jax.ShapeDtypeStruct((_NC * _NACC,), jnp.float32),
        mesh=mesh,
        scratch_types=[
            pltpu.VMEM((_RPT, _LANES), jnp.int32),
            pltpu.VMEM((_LANES,), jnp.float32),
            pltpu.VMEM((_LANES,), jnp.float32),
            pltpu.MemorySpace.VMEM_SHARED((_NACC,), jnp.float32),
            pltpu.SemaphoreType.DMA,
        ],
    )
    def deg_kernel(dst_hbm, out_hbm, didx, ones_v, zb, dacc, sem):
        c = lax.axis_index("c")
        s = lax.axis_index("s")
        wid = s * _NC + c

        for cb in range(_LANES // 16):
            ones_v[pl.ds(cb * 16, 16)] = jnp.ones((16,), jnp.float32)
        for cb in range(_LANES // 16):
            zb[pl.ds(cb * 16, 16)] = jnp.zeros((16,), jnp.float32)
        for k in range(_STRIPE // _LANES):
            pltpu.sync_copy(zb, dacc.at[pl.ds(s * _STRIPE + k * _LANES,
                                              _LANES)])
        plsc.subcore_barrier()
        pltpu.sync_copy(dst_hbm.at[pl.ds(wid * _RPT, _RPT)], didx)

        def body(j, carry):
            pltpu.async_copy(ones_v, dacc.at[didx.at[j]], sem, add=True)
            return carry
        lax.fori_loop(0, _RPT, body, 0)

        def drain(j, carry):
            pltpu.make_async_copy(ones_v, dacc.at[didx.at[j]], sem).wait()
            return carry
        lax.fori_loop(0, _RPT, drain, 0)
        plsc.subcore_barrier()
        pltpu.sync_copy(dacc.at[pl.ds(s * _STRIPE, _STRIPE)],
                        out_hbm.at[pl.ds(c * _NACC + s * _STRIPE, _STRIPE)])

    return deg_kernel(dst2d).reshape(_NC, _NACC)


def _sc_scatter(hws2, srcoff, dst2d):
    mesh = plsc.VectorSubcoreMesh(core_axis_name="c", subcore_axis_name="s")

    @functools.partial(
        pl.kernel,
        out_type=jax.ShapeDtypeStruct((_NC, _NACC, _HD), jnp.float32),
        mesh=mesh,
        scratch_types=[
            pltpu.VMEM((_RPC, _LANES), jnp.int32),
            pltpu.VMEM((_RPC, _LANES), jnp.int32),
            pltpu.VMEM((_LANES, _HD), jnp.float32),
            pltpu.VMEM((_LANES, _HD), jnp.float32),
            pltpu.VMEM((_LANES, _HD), jnp.float32),
            pltpu.VMEM((_LANES, _HD), jnp.float32),
            pltpu.VMEM((_LANES, _HD), jnp.float32),
            pltpu.MemorySpace.VMEM_SHARED((_NACC, _HD), jnp.float32),
            pltpu.SemaphoreType.DMA,
            pltpu.SemaphoreType.DMA,
        ],
        compiler_params=pltpu.CompilerParams(use_tc_tiling_on_sc=False),
    )
    def scat_kernel(hws_hbm, src_hbm, dst_hbm, out_hbm,
                    sidx, didx, rows_a, rows_b, rows_c, rows_d,
                    rows_e, acc, sem_a, sem_s):
        c = lax.axis_index("c")
        s = lax.axis_index("s")
        bufs = (rows_a, rows_b, rows_c, rows_d, rows_e)
        nbuf = len(bufs)

        def zr(i, carry):
            for cb in range(_HD // 16):
                rows_a[i, pl.ds(cb * 16, 16)] = jnp.zeros((16,), jnp.float32)
            return carry
        lax.fori_loop(0, _LANES, zr, 0)

        def zcopy(k, carry):
            pltpu.sync_copy(rows_a,
                            acc.at[pl.ds(s * _STRIPE + k * _LANES, _LANES)])
            return carry
        lax.fori_loop(0, _STRIPE // _LANES, zcopy, 0)
        plsc.subcore_barrier()
        pltpu.sync_copy(src_hbm.at[c, pl.ds(s * _RPC, _RPC)], sidx)
        pltpu.sync_copy(dst_hbm.at[pl.ds(s * _RPC, _RPC)], didx)

        for b in range(nbuf - 1):
            pltpu.async_copy(hws_hbm.at[sidx.at[b]], bufs[b], sem_a)

        def body(i, carry):
            for b in range(nbuf):
                j = nbuf * i + b
                bp = (b + nbuf - 1) % nbuf
                pltpu.make_async_copy(
                    hws_hbm.at[sidx.at[j]], bufs[b], sem_a).wait()
                pltpu.async_copy(bufs[b], acc.at[didx.at[j]], sem_s,
                                 add=True)

                @pl.when(j >= 1)
                def _retire():
                    pltpu.make_async_copy(
                        bufs[bp], acc.at[didx.at[j - 1]], sem_s).wait()

                @pl.when(j + nbuf - 1 < _RPC)
                def _reuse():
                    pltpu.async_copy(
                        hws_hbm.at[sidx.at[j + nbuf - 1]], bufs[bp], sem_a)
            return carry
        lax.fori_loop(0, _RPC // nbuf, body, 0)
        pltpu.make_async_copy(
            bufs[(_RPC - 1) % nbuf], acc.at[didx.at[_RPC - 1]], sem_s).wait()
        plsc.subcore_barrier()
        pltpu.sync_copy(acc.at[pl.ds(s * _STRIPE, _STRIPE)],
                        out_hbm.at[c, pl.ds(s * _STRIPE, _STRIPE)])

    return scat_kernel(hws2, srcoff, dst2d)


def _recons_body(sp_ref, hws_ref, dinv_ref, bg_ref, o_ref, h_scr):
    @pl.when(pl.program_id(0) == 0)
    def _build_h():
        sfull = jnp.concatenate([sp_ref[0, :N, :], sp_ref[1, :N, :]], axis=1)
        hws = jnp.concatenate([hws_ref[0], hws_ref[1]], axis=1)
        v = dinv_ref[...] * (sfull + hws) + bg_ref[...]
        h_scr[...] = jnp.where(v >= 0, v, 0.01 * v)

    i = pl.program_id(0)
    hb = h_scr[pl.ds(i * _TM_REC, _TM_REC), :]
    logits = lax.dot_general(
        hb, h_scr[...],
        (((1,), (1,)), ((), ())),
        preferred_element_type=jnp.float32)
    o_ref[...] = 0.5 * jnp.tanh(0.5 * logits) + 0.5


def _recons(spart, hws2s, dinv, bg):
    return pl.pallas_call(
        _recons_body,
        grid=(N // _TM_REC,),
        in_specs=[
            pl.BlockSpec((_NC, _NACC, _HD), lambda i: (0, 0, 0)),
            pl.BlockSpec((_NC, N, _HD), lambda i: (0, 0, 0)),
            pl.BlockSpec((N, 1), lambda i: (0, 0)),
            pl.BlockSpec((1, D), lambda i: (0, 0)),
        ],
        out_specs=pl.BlockSpec((_TM_REC, N), lambda i: (i, 0)),
        out_shape=jax.ShapeDtypeStruct((N, N), jnp.float32),
        scratch_shapes=[pltpu.VMEM((N, D), jnp.float32)],
        compiler_params=pltpu.CompilerParams(
            dimension_semantics=("arbitrary",)),
    )(spart, hws2s, dinv, bg.reshape(1, D))


def kernel(x, edge_index, Wd, bd, Wg, bg):
    src = edge_index[0]
    dst = edge_index[1]
    pad = _EP - E
    ar = jnp.arange(pad, dtype=jnp.int32)
    src2d = jnp.concatenate([src, ar % 64]).reshape(_IDX_ROWS, _LANES)
    dst2d = jnp.concatenate([dst, N + (ar % 16)]).reshape(_IDX_ROWS, _LANES)

    degp = _sc_deg(dst2d)
    d0 = degp[0, :N].reshape(N, 1)
    d1 = degp[1, :N].reshape(N, 1)
    hws2s, dinv = _dense(x, Wd, bd, Wg, d0, d1)
    hws2 = hws2s.reshape(_NC * N, _HD)
    srcoff = jnp.stack([src2d, src2d + N])
    spart = _sc_scatter(hws2, srcoff, dst2d)
    return _recons(spart, hws2s, dinv, bg)
